# baseline (speedup 1.0000x reference)
import jax
import jax.numpy as jnp
from jax.experimental import pallas as pl

TEM_NUM = 3


def _fused_kernel(pts_ref, tid_ref, w1_ref, w2_ref, out_ref):
    pts = pts_ref[0]
    tids = tid_ref[0]
    w1 = w1_ref[...]
    w2 = w2_ref[...]

    h = jnp.maximum(jnp.dot(pts, w1, preferred_element_type=jnp.float32), 0.0)
    h = jnp.maximum(jnp.dot(h, w2, preferred_element_type=jnp.float32), 0.0)

    at = jnp.abs(tids)
    parts = []
    for t in range(1, TEM_NUM):
        m = (at == t).astype(jnp.float32)
        s = jnp.dot(m, h, preferred_element_type=jnp.float32)
        c = jnp.maximum(jnp.sum(m), 1.0)
        parts.append(s[0] / c)
    out_ref[0, 0, :] = jnp.concatenate(parts, axis=0)


def kernel(points, time_ids, W1, W2):
    B, N, D = points.shape
    H = W1.shape[1]
    n_t = TEM_NUM - 1
    tids3 = time_ids.reshape(B, 1, N)

    out = pl.pallas_call(
        _fused_kernel,
        grid=(B,),
        in_specs=[
            pl.BlockSpec((1, N, D), lambda b: (b, 0, 0)),
            pl.BlockSpec((1, 1, N), lambda b: (b, 0, 0)),
            pl.BlockSpec((D, H), lambda b: (0, 0)),
            pl.BlockSpec((H, H), lambda b: (0, 0)),
        ],
        out_specs=pl.BlockSpec((1, 1, n_t * H), lambda b: (b, 0, 0)),
        out_shape=jax.ShapeDtypeStruct((B, 1, n_t * H), jnp.float32),
    )(points, tids3, W1, W2)

    return out.reshape(B, n_t, H).transpose(1, 0, 2)


# speedup vs baseline: 1.0303x; 1.0303x over previous
import jax
import jax.numpy as jnp
from jax.experimental import pallas as pl

TEM_NUM = 3


def _fused_kernel(pts_ref, tid_ref, w1_ref, w2_ref, out_ref):
    pts = pts_ref[0].astype(jnp.bfloat16)
    tids = tid_ref[0]
    w1 = w1_ref[...].astype(jnp.bfloat16)
    w2 = w2_ref[...].astype(jnp.bfloat16)

    h = jnp.maximum(jnp.dot(pts, w1, preferred_element_type=jnp.float32), 0.0)
    h = jnp.maximum(
        jnp.dot(h.astype(jnp.bfloat16), w2, preferred_element_type=jnp.float32),
        0.0,
    ).astype(jnp.bfloat16)

    at = jnp.abs(tids)
    masks = jnp.concatenate(
        [(at == t).astype(jnp.bfloat16) for t in range(1, TEM_NUM)], axis=0
    )
    sums = jnp.dot(masks, h, preferred_element_type=jnp.float32)
    cnts = jnp.maximum(
        jnp.sum(masks.astype(jnp.float32), axis=1, keepdims=True), 1.0
    )
    out_ref[0, 0, :] = (sums / cnts).reshape(-1)


def kernel(points, time_ids, W1, W2):
    B, N, D = points.shape
    H = W1.shape[1]
    n_t = TEM_NUM - 1
    tids3 = time_ids.reshape(B, 1, N)

    out = pl.pallas_call(
        _fused_kernel,
        grid=(B,),
        in_specs=[
            pl.BlockSpec((1, N, D), lambda b: (b, 0, 0)),
            pl.BlockSpec((1, 1, N), lambda b: (b, 0, 0)),
            pl.BlockSpec((D, H), lambda b: (0, 0)),
            pl.BlockSpec((H, H), lambda b: (0, 0)),
        ],
        out_specs=pl.BlockSpec((1, 1, n_t * H), lambda b: (b, 0, 0)),
        out_shape=jax.ShapeDtypeStruct((B, 1, n_t * H), jnp.float32),
    )(points, tids3, W1, W2)

    return out.reshape(B, n_t, H).transpose(1, 0, 2)


# speedup vs baseline: 1.4784x; 1.4348x over previous
import jax
import jax.numpy as jnp
from jax import lax
from jax.experimental import pallas as pl

TEM_NUM = 3


def _fused_kernel(pts_ref, tid_ref, w1t_ref, w2t_ref, out_ref):
    ptsT = pts_ref[0].astype(jnp.bfloat16)
    tids = tid_ref[0]
    w1t = w1t_ref[...].astype(jnp.bfloat16)
    w2t = w2t_ref[...].astype(jnp.bfloat16)

    h = jnp.maximum(jnp.dot(w1t, ptsT, preferred_element_type=jnp.float32), 0.0)
    h = jnp.maximum(
        jnp.dot(w2t, h.astype(jnp.bfloat16), preferred_element_type=jnp.float32),
        0.0,
    ).astype(jnp.bfloat16)

    at = jnp.abs(tids)
    masks = jnp.concatenate(
        [(at == t).astype(jnp.bfloat16) for t in range(1, TEM_NUM)], axis=0
    )
    sums = lax.dot_general(
        h, masks, (((1,), (1,)), ((), ())),
        preferred_element_type=jnp.float32,
    )
    for t in range(1, TEM_NUM):
        c = jnp.maximum(jnp.sum(masks[t - 1 : t, :].astype(jnp.float32)), 1.0)
        out_ref[0, :, t - 1 : t] = sums[:, t - 1 : t] / c


def kernel(points, time_ids, W1, W2):
    B, N, D = points.shape
    H = W1.shape[1]
    n_t = TEM_NUM - 1
    ptsT = points.transpose(0, 2, 1)
    tids3 = time_ids.reshape(B, 1, N)

    out = pl.pallas_call(
        _fused_kernel,
        grid=(B,),
        in_specs=[
            pl.BlockSpec((1, D, N), lambda b: (b, 0, 0)),
            pl.BlockSpec((1, 1, N), lambda b: (b, 0, 0)),
            pl.BlockSpec((H, D), lambda b: (0, 0)),
            pl.BlockSpec((H, H), lambda b: (0, 0)),
        ],
        out_specs=pl.BlockSpec((1, H, n_t), lambda b: (b, 0, 0)),
        out_shape=jax.ShapeDtypeStruct((B, H, n_t), jnp.float32),
    )(ptsT, tids3, W1.T, W2.T)

    return out.transpose(2, 0, 1)


# speedup vs baseline: 1.9898x; 1.3460x over previous
import jax
import jax.numpy as jnp
from jax import lax
from jax.experimental import pallas as pl

TEM_NUM = 3
BPS = 4


def _fused_kernel(pts_ref, tid_ref, w1t_ref, w2t_ref, out_ref):
    NB = pts_ref.shape[1]
    N = NB // BPS
    ptsT = pts_ref[...].astype(jnp.bfloat16)
    at = jnp.abs(tid_ref[...])
    w1t = w1t_ref[...].astype(jnp.bfloat16)
    w2t = w2t_ref[...].astype(jnp.bfloat16)

    h = jnp.maximum(
        jnp.dot(w1t, ptsT, preferred_element_type=jnp.float32), 0.0
    ).astype(jnp.bfloat16)
    h = jnp.maximum(
        jnp.dot(w2t, h, preferred_element_type=jnp.float32), 0.0
    ).astype(jnp.bfloat16)

    n_t = TEM_NUM - 1
    tvec = lax.broadcasted_iota(jnp.int32, (n_t, N), 0) + 1
    for i in range(BPS):
        at_i = at[:, i * N : (i + 1) * N]
        masks = (jnp.broadcast_to(at_i, (n_t, N)) == tvec).astype(jnp.bfloat16)
        sums = lax.dot_general(
            h[:, i * N : (i + 1) * N], masks, (((1,), (1,)), ((), ())),
            preferred_element_type=jnp.float32,
        )
        for t in range(1, TEM_NUM):
            c = jnp.maximum(jnp.sum((at_i == t).astype(jnp.float32)), 1.0)
            out_ref[i, :, t - 1 : t] = sums[:, t - 1 : t] / c


def kernel(points, time_ids, W1, W2):
    B, N, D = points.shape
    H = W1.shape[1]
    n_t = TEM_NUM - 1
    ptsT = points.transpose(2, 0, 1).reshape(D, B * N)
    tids2 = time_ids.reshape(1, B * N)

    out = pl.pallas_call(
        _fused_kernel,
        grid=(B // BPS,),
        in_specs=[
            pl.BlockSpec((D, BPS * N), lambda g: (0, g)),
            pl.BlockSpec((1, BPS * N), lambda g: (0, g)),
            pl.BlockSpec((H, D), lambda g: (0, 0)),
            pl.BlockSpec((H, H), lambda g: (0, 0)),
        ],
        out_specs=pl.BlockSpec((BPS, H, n_t), lambda g: (g, 0, 0)),
        out_shape=jax.ShapeDtypeStruct((B, H, n_t), jnp.float32),
    )(ptsT, tids2, W1.T, W2.T)

    return out.transpose(2, 0, 1)
